# baseline (speedup 1.0000x reference)
import jax
import jax.numpy as jnp
from jax import lax
from jax.experimental import pallas as pl
from jax.experimental.pallas import tpu as pltpu
from jax.experimental.pallas import tpu_sc as plsc

B = 4096
DENSE = 13
NSPARSE = 26
HIST = 50
VOCAB = 100000
ED = 32
OUT_D = DENSE + NSPARSE * ED + ED

VC = 3840
NCH = 25
TAIL0 = NCH * VC
TAILW = 3968
LAST0 = TAIL0 + TAILW
TAILP = 4096
VPLANE = 102400
NUNIT = NSPARSE * (ED // 8)


def _varlen_body(vidx_hbm, dense_hbm, vt_hbm, odense_hbm, ovar_hbm,
                 plane_v, vch_v, col_v, sem):
  wid = lax.axis_index("c") * 16 + lax.axis_index("s")

  @pl.when(wid < DENSE)
  def _():
    pltpu.sync_copy(dense_hbm.at[pl.ds(wid * B, B)], col_v)
    pltpu.sync_copy(col_v, odense_hbm.at[pl.ds(wid * B, B)])

  pltpu.sync_copy(vt_hbm.at[pl.ds(wid * VPLANE, VPLANE)], plane_v)

  def vchunk_body(c, carry):
    pltpu.sync_copy(vidx_hbm.at[pl.ds(c * (HIST * 256), HIST * 256)],
                    vch_v)

    def vgrp_body(g, carry2):
      def hbody(h, acc):
        idx = vch_v[pl.ds(h * 256 + g * 16, 16)]
        return acc + plsc.load_gather(plane_v, [idx])
      acc = lax.fori_loop(0, HIST, hbody, jnp.zeros((16,), jnp.float32),
                          unroll=5)
      col_v[pl.ds(c * 256 + g * 16, 16)] = acc * (1.0 / HIST)
      return carry2

    lax.fori_loop(0, 16, vgrp_body, 0)
    return carry

  lax.fori_loop(0, B // 256, vchunk_body, 0)
  pltpu.sync_copy(col_v, ovar_hbm.at[pl.ds(wid * B, B)])


def _sparse_body(stT_hbm, tail_hbm, offs_hbm, vs_hbm, out_hbm,
                 slab_a, slab_b, vs_v, offs_v, sem_a, sem_b, *cols):
  wid = lax.axis_index("c") * 16 + lax.axis_index("s")
  if True:
    def slab_cp(f, j, c, buf, s):
      return pltpu.make_async_copy(
          stT_hbm.at[f, pl.ds(j * 8, 8), pl.ds(c * VC, VC)],
          buf.at[:, pl.ds(0, VC)], s)

    def tail_cp(f, j, buf, s):
      return pltpu.make_async_copy(
          stT_hbm.at[f, pl.ds(j * 8, 8), pl.ds(TAIL0, TAILW)],
          buf.at[:, pl.ds(0, TAILW)], s)

    def last_cp(f, j, buf, s):
      return pltpu.make_async_copy(
          tail_hbm.at[f, pl.ds(j * 8, 8)], buf.at[:, pl.ds(0, 128)], s)

    def do_unit(u):
      f = lax.rem(u, NSPARSE)
      j = lax.div(u, NSPARSE)
      pltpu.sync_copy(vs_hbm.at[pl.ds(f * B, B)], vs_v.at[pl.ds(0, B)])
      pltpu.sync_copy(offs_hbm.at[pl.ds(f * 48, 48)],
                      offs_v.at[pl.ds(0, 48)])

      def process(buf, c, base):
        se = offs_v[pl.ds(c, 16)]
        start, end = se[0], se[1]

        def grp(g, carry2):
          pos = start + g * 16
          p16 = vs_v[pl.ds(pos, 16)]
          v16 = lax.shift_right_logical(p16, 12)
          b16 = lax.bitwise_and(p16, 4095)
          mask = (pos + lax.iota(jnp.int32, 16)) < end
          vrel = v16 - base
          for e in range(8):
            vals = plsc.load_gather(
                buf, [jnp.full((16,), e, jnp.int32), vrel], mask=mask)
            plsc.store_scatter(cols[e], [b16], vals, mask=mask)
          return carry2

        lax.fori_loop(0, lax.div(end - start + 15, 16), grp, 0)

      slab_cp(f, j, 0, slab_a, sem_a).start()

      def pair(p, carry):
        c0 = 2 * p
        slab_cp(f, j, c0 + 1, slab_b, sem_b).start()
        slab_cp(f, j, c0, slab_a, sem_a).wait()
        process(slab_a, c0, c0 * VC)
        slab_cp(f, j, c0 + 2, slab_a, sem_a).start()
        slab_cp(f, j, c0 + 1, slab_b, sem_b).wait()
        process(slab_b, c0 + 1, (c0 + 1) * VC)
        return carry

      lax.fori_loop(0, (NCH - 1) // 2, pair, 0)
      tail_cp(f, j, slab_b, sem_b).start()
      slab_cp(f, j, NCH - 1, slab_a, sem_a).wait()
      process(slab_a, NCH - 1, (NCH - 1) * VC)
      last_cp(f, j, slab_a, sem_a).start()
      tail_cp(f, j, slab_b, sem_b).wait()
      process(slab_b, NCH, TAIL0)
      last_cp(f, j, slab_a, sem_a).wait()
      process(slab_a, NCH + 1, LAST0)

      for e in range(8):
        pltpu.sync_copy(
            cols[e],
            out_hbm.at[pl.ds((f * ED + j * 8 + e) * B, B)])

    def unit_loop(k, carry):
      do_unit(wid + k * 32)
      return carry

    lax.fori_loop(0, 3, unit_loop, 0)

    @pl.when(lax.rem(wid, 4) == 0)
    def _():
      do_unit(96 + lax.div(wid, 4))


@jax.jit
def kernel(x, sparse_tables, varlen_table):
  xT = x.T
  sidxT = xT[DENSE:DENSE + NSPARSE].astype(jnp.int32)
  vidx_ch = (xT[DENSE + NSPARSE:].astype(jnp.int32)
             .reshape(HIST, 16, 256).transpose(1, 0, 2).reshape(-1))
  dense_flat = xT[:DENSE].reshape(-1)
  stT = sparse_tables.transpose(0, 2, 1)
  vt_pad = jnp.pad(varlen_table.T,
                   ((0, 0), (0, VPLANE - VOCAB))).reshape(-1)
  tail_aux = jnp.pad(stT[:, :, LAST0:],
                     ((0, 0), (0, 0), (0, 128 - (VOCAB - LAST0))))

  binit = jnp.broadcast_to(jnp.arange(B, dtype=jnp.int32)[None, :],
                           (NSPARSE, B))
  keys = jnp.sort(sidxT * B + binit, axis=1)
  bid = jnp.where(sidxT < TAIL0, sidxT // VC,
                  jnp.where(sidxT < LAST0, NCH, NCH + 1))
  counts = jnp.sum(
      (bid[:, :, None] == jnp.arange(NCH + 2, dtype=jnp.int32)[None, None, :]
       ).astype(jnp.int32), axis=1)
  offs = jnp.concatenate(
      [jnp.zeros((NSPARSE, 1), jnp.int32),
       jnp.cumsum(counts, axis=1).astype(jnp.int32),
       jnp.zeros((NSPARSE, 48 - (NCH + 3)), jnp.int32)], axis=1)

  cparams = pltpu.CompilerParams(
      use_tc_tiling_on_sc=True, needs_layout_passes=False)
  mesh = plsc.VectorSubcoreMesh(core_axis_name="c", subcore_axis_name="s")
  run_varlen = pl.kernel(
      _varlen_body,
      out_type=(jax.ShapeDtypeStruct((DENSE * B,), jnp.float32),
                jax.ShapeDtypeStruct((ED * B,), jnp.float32)),
      mesh=mesh,
      compiler_params=cparams,
      scratch_types=[
          pltpu.VMEM((VPLANE,), jnp.float32),
          pltpu.VMEM((HIST * 256,), jnp.int32),
          pltpu.VMEM((B,), jnp.float32),
          pltpu.SemaphoreType.DMA,
      ],
  )
  run_sparse = pl.kernel(
      _sparse_body,
      out_type=jax.ShapeDtypeStruct((NSPARSE * ED * B,), jnp.float32),
      mesh=mesh,
      compiler_params=cparams,
      scratch_types=[
          pltpu.VMEM((8, TAILP), jnp.float32),
          pltpu.VMEM((8, TAILP), jnp.float32),
          pltpu.VMEM((B + 32,), jnp.int32),
          pltpu.VMEM((64,), jnp.int32),
          pltpu.SemaphoreType.DMA,
          pltpu.SemaphoreType.DMA,
          *[pltpu.VMEM((B,), jnp.float32) for _ in range(8)],
      ],
  )
  o_dense, o_var = run_varlen(vidx_ch, dense_flat, vt_pad)
  o_sparse = run_sparse(stT, tail_aux, offs.reshape(-1), keys.reshape(-1))
  out_flat = jnp.concatenate([o_dense, o_sparse, o_var])
  return out_flat.reshape(OUT_D, B).T


# speedup vs baseline: 1.1255x; 1.1255x over previous
import jax
import jax.numpy as jnp
from jax import lax
from jax.experimental import pallas as pl
from jax.experimental.pallas import tpu as pltpu
from jax.experimental.pallas import tpu_sc as plsc

B = 4096
DENSE = 13
NSPARSE = 26
HIST = 50
VOCAB = 100000
ED = 32
OUT_D = DENSE + NSPARSE * ED + ED

VC = 3840
NCH = 25
TAIL0 = NCH * VC
TAILW = 3968
LAST0 = TAIL0 + TAILW
TAILP = 4096
VPLANE = 102400
NUNIT = NSPARSE * (ED // 8)


def _varlen_body(vidx_hbm, dense_hbm, vt_hbm, odense_hbm, ovar_hbm,
                 plane_v, vch_v, col_v, sem):
  wid = lax.axis_index("c") * 16 + lax.axis_index("s")

  @pl.when(wid < DENSE)
  def _():
    pltpu.sync_copy(dense_hbm.at[pl.ds(wid * B, B)], col_v)
    pltpu.sync_copy(col_v, odense_hbm.at[pl.ds(wid * B, B)])

  pltpu.sync_copy(vt_hbm.at[pl.ds(wid * VPLANE, VPLANE)], plane_v)

  def vchunk_body(c, carry):
    pltpu.sync_copy(vidx_hbm.at[pl.ds(c * (HIST * 256), HIST * 256)],
                    vch_v)

    def vgrp_body(g, carry2):
      def hbody(h, acc):
        idx = vch_v[pl.ds(h * 256 + g * 16, 16)]
        return acc + plsc.load_gather(plane_v, [idx])
      acc = lax.fori_loop(0, HIST, hbody, jnp.zeros((16,), jnp.float32),
                          unroll=5)
      col_v[pl.ds(c * 256 + g * 16, 16)] = acc * (1.0 / HIST)
      return carry2

    lax.fori_loop(0, 16, vgrp_body, 0)
    return carry

  lax.fori_loop(0, B // 256, vchunk_body, 0)
  pltpu.sync_copy(col_v, ovar_hbm.at[pl.ds(wid * B, B)])


def _sparse_body(stT_hbm, tail_hbm, offs_hbm, vs_hbm, order_hbm, out_hbm,
                 slab_a, slab_b, vs_v, offs_v, sem_a, sem_b, *cols):
  del order_hbm
  wid = lax.axis_index("c") * 16 + lax.axis_index("s")
  if True:
    def slab_cp(f, j, c, buf, s):
      return pltpu.make_async_copy(
          stT_hbm.at[f, pl.ds(j * 8, 8), pl.ds(c * VC, VC)],
          buf.at[:, pl.ds(0, VC)], s)

    def tail_cp(f, j, buf, s):
      return pltpu.make_async_copy(
          stT_hbm.at[f, pl.ds(j * 8, 8), pl.ds(TAIL0, TAILW)],
          buf.at[:, pl.ds(0, TAILW)], s)

    def last_cp(f, j, buf, s):
      return pltpu.make_async_copy(
          tail_hbm.at[f, pl.ds(j * 8, 8)], buf.at[:, pl.ds(0, 128)], s)

    def do_unit(u):
      f = lax.rem(u, NSPARSE)
      j = lax.div(u, NSPARSE)
      pltpu.sync_copy(vs_hbm.at[pl.ds(f * B, B)], vs_v.at[pl.ds(0, B)])
      pltpu.sync_copy(offs_hbm.at[pl.ds(f * 48, 48)],
                      offs_v.at[pl.ds(0, 48)])

      def process(buf, c, base):
        se = offs_v[pl.ds(c, 16)]
        start, end = se[0], se[1]

        def grp(g, carry2):
          pos = start + g * 16
          p16 = vs_v[pl.ds(pos, 16)]
          v16 = lax.shift_right_logical(p16, 12)
          b16 = lax.bitwise_and(p16, 4095)
          mask = (pos + lax.iota(jnp.int32, 16)) < end
          vrel = v16 - base
          for e in range(8):
            vals = plsc.load_gather(
                buf, [jnp.full((16,), e, jnp.int32), vrel], mask=mask)
            plsc.store_scatter(cols[e], [b16], vals, mask=mask)
          return carry2

        lax.fori_loop(0, lax.div(end - start + 15, 16), grp, 0)

      slab_cp(f, j, 0, slab_a, sem_a).start()

      def pair(p, carry):
        c0 = 2 * p
        slab_cp(f, j, c0 + 1, slab_b, sem_b).start()
        slab_cp(f, j, c0, slab_a, sem_a).wait()
        process(slab_a, c0, c0 * VC)
        slab_cp(f, j, c0 + 2, slab_a, sem_a).start()
        slab_cp(f, j, c0 + 1, slab_b, sem_b).wait()
        process(slab_b, c0 + 1, (c0 + 1) * VC)
        return carry

      lax.fori_loop(0, (NCH - 1) // 2, pair, 0)
      tail_cp(f, j, slab_b, sem_b).start()
      slab_cp(f, j, NCH - 1, slab_a, sem_a).wait()
      process(slab_a, NCH - 1, (NCH - 1) * VC)
      last_cp(f, j, slab_a, sem_a).start()
      tail_cp(f, j, slab_b, sem_b).wait()
      process(slab_b, NCH, TAIL0)
      last_cp(f, j, slab_a, sem_a).wait()
      process(slab_a, NCH + 1, LAST0)

      for e in range(8):
        pltpu.sync_copy(
            cols[e],
            out_hbm.at[pl.ds((f * ED + j * 8 + e) * B, B)])

    def unit_loop(k, carry):
      do_unit(wid + k * 32)
      return carry

    lax.fori_loop(0, 3, unit_loop, 0)

    @pl.when(lax.rem(wid, 4) == 0)
    def _():
      do_unit(96 + lax.div(wid, 4))


@jax.jit
def kernel(x, sparse_tables, varlen_table):
  xT = x.T
  sidxT = xT[DENSE:DENSE + NSPARSE].astype(jnp.int32)
  vidx_ch = (xT[DENSE + NSPARSE:].astype(jnp.int32)
             .reshape(HIST, 16, 256).transpose(1, 0, 2).reshape(-1))
  dense_flat = xT[:DENSE].reshape(-1)
  stT = sparse_tables.transpose(0, 2, 1)
  vt_pad = jnp.pad(varlen_table.T,
                   ((0, 0), (0, VPLANE - VOCAB))).reshape(-1)
  tail_aux = jnp.pad(stT[:, :, LAST0:],
                     ((0, 0), (0, 0), (0, 128 - (VOCAB - LAST0))))

  binit = jnp.broadcast_to(jnp.arange(B, dtype=jnp.int32)[None, :],
                           (NSPARSE, B))
  keys = jnp.sort(sidxT * B + binit, axis=1)
  bid = jnp.where(sidxT < TAIL0, sidxT // VC,
                  jnp.where(sidxT < LAST0, NCH, NCH + 1))
  counts = jnp.sum(
      (bid[:, :, None] == jnp.arange(NCH + 2, dtype=jnp.int32)[None, None, :]
       ).astype(jnp.int32), axis=1)
  offs = jnp.concatenate(
      [jnp.zeros((NSPARSE, 1), jnp.int32),
       jnp.cumsum(counts, axis=1).astype(jnp.int32),
       jnp.zeros((NSPARSE, 48 - (NCH + 3)), jnp.int32)], axis=1)

  cparams = pltpu.CompilerParams(
      use_tc_tiling_on_sc=True, needs_layout_passes=False)
  mesh = plsc.VectorSubcoreMesh(core_axis_name="c", subcore_axis_name="s")
  run_varlen = pl.kernel(
      _varlen_body,
      out_type=(jax.ShapeDtypeStruct((DENSE * B,), jnp.float32),
                jax.ShapeDtypeStruct((ED * B,), jnp.float32)),
      mesh=mesh,
      compiler_params=cparams,
      scratch_types=[
          pltpu.VMEM((VPLANE,), jnp.float32),
          pltpu.VMEM((HIST * 256,), jnp.int32),
          pltpu.VMEM((B,), jnp.float32),
          pltpu.SemaphoreType.DMA,
      ],
  )
  run_sparse = pl.kernel(
      _sparse_body,
      out_type=jax.ShapeDtypeStruct((NSPARSE * ED * B,), jnp.float32),
      mesh=mesh,
      compiler_params=cparams,
      scratch_types=[
          pltpu.VMEM((8, TAILP), jnp.float32),
          pltpu.VMEM((8, TAILP), jnp.float32),
          pltpu.VMEM((B + 32,), jnp.int32),
          pltpu.VMEM((64,), jnp.int32),
          pltpu.SemaphoreType.DMA,
          pltpu.SemaphoreType.DMA,
          *[pltpu.VMEM((B,), jnp.float32) for _ in range(8)],
      ],
  )
  o_dense, o_var = run_varlen(vidx_ch, dense_flat, vt_pad)
  o_sparse = run_sparse(stT, tail_aux, offs.reshape(-1), keys.reshape(-1),
                        o_dense)
  out_flat = jnp.concatenate([o_dense, o_sparse, o_var])
  return out_flat.reshape(OUT_D, B).T


# speedup vs baseline: 1.1506x; 1.0223x over previous
import jax
import jax.numpy as jnp
from jax import lax
from jax.experimental import pallas as pl
from jax.experimental.pallas import tpu as pltpu
from jax.experimental.pallas import tpu_sc as plsc

B = 4096
DENSE = 13
NSPARSE = 26
HIST = 50
VOCAB = 100000
ED = 32
OUT_D = DENSE + NSPARSE * ED + ED

VC = 3840
NCH = 25
TAIL0 = NCH * VC
TAILW = 3968
LAST0 = TAIL0 + TAILW
TAILP = 4096
VPLANE = 102400
NUNIT = NSPARSE * (ED // 8)


def _varlen_body(vidx_hbm, dense_hbm, vt_hbm, out_hbm,
                 plane_v, vch_v, col_v, sem):
  wid = lax.axis_index("c") * 16 + lax.axis_index("s")

  @pl.when(wid < DENSE)
  def _():
    pltpu.sync_copy(dense_hbm.at[pl.ds(wid * B, B)], col_v)
    pltpu.sync_copy(col_v, out_hbm.at[pl.ds(wid * B, B)])

  pltpu.sync_copy(vt_hbm.at[pl.ds(wid * VPLANE, VPLANE)], plane_v)

  def vchunk_body(c, carry):
    pltpu.sync_copy(vidx_hbm.at[pl.ds(c * (HIST * 256), HIST * 256)],
                    vch_v)

    def vgrp_body(g, carry2):
      def hbody(h, acc):
        idx = vch_v[pl.ds(h * 256 + g * 16, 16)]
        return acc + plsc.load_gather(plane_v, [idx])
      acc = lax.fori_loop(0, HIST, hbody, jnp.zeros((16,), jnp.float32),
                          unroll=5)
      col_v[pl.ds(c * 256 + g * 16, 16)] = acc * (1.0 / HIST)
      return carry2

    lax.fori_loop(0, 16, vgrp_body, 0)
    return carry

  lax.fori_loop(0, B // 256, vchunk_body, 0)
  pltpu.sync_copy(col_v,
                  out_hbm.at[pl.ds((DENSE + NSPARSE * ED + wid) * B, B)])


def _sparse_body(stT_hbm, tail_hbm, offs_hbm, vs_hbm, out_hbm,
                 slab_a, slab_b, vs_v, offs_v, sem_a, sem_b, *cols):
  wid = lax.axis_index("c") * 16 + lax.axis_index("s")
  if True:
    def slab_cp(f, j, c, buf, s):
      return pltpu.make_async_copy(
          stT_hbm.at[f, pl.ds(j * 8, 8), pl.ds(c * VC, VC)],
          buf.at[:, pl.ds(0, VC)], s)

    def tail_cp(f, j, buf, s):
      return pltpu.make_async_copy(
          stT_hbm.at[f, pl.ds(j * 8, 8), pl.ds(TAIL0, TAILW)],
          buf.at[:, pl.ds(0, TAILW)], s)

    def last_cp(f, j, buf, s):
      return pltpu.make_async_copy(
          tail_hbm.at[f, pl.ds(j * 8, 8)], buf.at[:, pl.ds(0, 128)], s)

    def do_unit(u):
      f = lax.rem(u, NSPARSE)
      j = lax.div(u, NSPARSE)
      pltpu.sync_copy(vs_hbm.at[pl.ds(f * B, B)], vs_v.at[pl.ds(0, B)])
      pltpu.sync_copy(offs_hbm.at[pl.ds(f * 48, 48)],
                      offs_v.at[pl.ds(0, 48)])

      def process(buf, c, base):
        se = offs_v[pl.ds(c, 16)]
        start, end = se[0], se[1]

        def grp(g, carry2):
          pos = start + g * 16
          p16 = vs_v[pl.ds(pos, 16)]
          v16 = lax.shift_right_logical(p16, 12)
          b16 = lax.bitwise_and(p16, 4095)
          mask = (pos + lax.iota(jnp.int32, 16)) < end
          vrel = v16 - base
          for e in range(8):
            vals = plsc.load_gather(
                buf, [jnp.full((16,), e, jnp.int32), vrel], mask=mask)
            plsc.store_scatter(cols[e], [b16], vals, mask=mask)
          return carry2

        lax.fori_loop(0, lax.div(end - start + 15, 16), grp, 0)

      slab_cp(f, j, 0, slab_a, sem_a).start()

      def pair(p, carry):
        c0 = 2 * p
        slab_cp(f, j, c0 + 1, slab_b, sem_b).start()
        slab_cp(f, j, c0, slab_a, sem_a).wait()
        process(slab_a, c0, c0 * VC)
        slab_cp(f, j, c0 + 2, slab_a, sem_a).start()
        slab_cp(f, j, c0 + 1, slab_b, sem_b).wait()
        process(slab_b, c0 + 1, (c0 + 1) * VC)
        return carry

      lax.fori_loop(0, (NCH - 1) // 2, pair, 0)
      tail_cp(f, j, slab_b, sem_b).start()
      slab_cp(f, j, NCH - 1, slab_a, sem_a).wait()
      process(slab_a, NCH - 1, (NCH - 1) * VC)
      last_cp(f, j, slab_a, sem_a).start()
      tail_cp(f, j, slab_b, sem_b).wait()
      process(slab_b, NCH, TAIL0)
      last_cp(f, j, slab_a, sem_a).wait()
      process(slab_a, NCH + 1, LAST0)

      for e in range(8):
        pltpu.sync_copy(
            cols[e],
            out_hbm.at[pl.ds((DENSE + f * ED + j * 8 + e) * B, B)])

    def unit_loop(k, carry):
      do_unit(wid + k * 32)
      return carry

    lax.fori_loop(0, 3, unit_loop, 0)

    @pl.when(lax.rem(wid, 4) == 0)
    def _():
      do_unit(96 + lax.div(wid, 4))


@jax.jit
def kernel(x, sparse_tables, varlen_table):
  xT = x.T
  sidxT = xT[DENSE:DENSE + NSPARSE].astype(jnp.int32)
  vidx_ch = (xT[DENSE + NSPARSE:].astype(jnp.int32)
             .reshape(HIST, 16, 256).transpose(1, 0, 2).reshape(-1))
  dense_flat = xT[:DENSE].reshape(-1)
  stT = sparse_tables.transpose(0, 2, 1)
  vt_pad = jnp.pad(varlen_table.T,
                   ((0, 0), (0, VPLANE - VOCAB))).reshape(-1)
  tail_aux = jnp.pad(stT[:, :, LAST0:],
                     ((0, 0), (0, 0), (0, 128 - (VOCAB - LAST0))))

  binit = jnp.broadcast_to(jnp.arange(B, dtype=jnp.int32)[None, :],
                           (NSPARSE, B))
  keys = jnp.sort(sidxT * B + binit, axis=1)
  bid = jnp.where(sidxT < TAIL0, sidxT // VC,
                  jnp.where(sidxT < LAST0, NCH, NCH + 1))
  counts = jnp.sum(
      (bid[:, :, None] == jnp.arange(NCH + 2, dtype=jnp.int32)[None, None, :]
       ).astype(jnp.int32), axis=1)
  offs = jnp.concatenate(
      [jnp.zeros((NSPARSE, 1), jnp.int32),
       jnp.cumsum(counts, axis=1).astype(jnp.int32),
       jnp.zeros((NSPARSE, 48 - (NCH + 3)), jnp.int32)], axis=1)

  cparams = pltpu.CompilerParams(
      use_tc_tiling_on_sc=True, needs_layout_passes=False)
  mesh = plsc.VectorSubcoreMesh(core_axis_name="c", subcore_axis_name="s")
  run_varlen = pl.kernel(
      _varlen_body,
      out_type=(),
      mesh=mesh,
      compiler_params=cparams,
      scratch_types=[
          pltpu.VMEM((VPLANE,), jnp.float32),
          pltpu.VMEM((HIST * 256,), jnp.int32),
          pltpu.VMEM((B,), jnp.float32),
          pltpu.SemaphoreType.DMA,
      ],
  )
  run_sparse = pl.kernel(
      _sparse_body,
      out_type=(),
      mesh=mesh,
      compiler_params=cparams,
      scratch_types=[
          pltpu.VMEM((8, TAILP), jnp.float32),
          pltpu.VMEM((8, TAILP), jnp.float32),
          pltpu.VMEM((B + 32,), jnp.int32),
          pltpu.VMEM((64,), jnp.int32),
          pltpu.SemaphoreType.DMA,
          pltpu.SemaphoreType.DMA,
          *[pltpu.VMEM((B,), jnp.float32) for _ in range(8)],
      ],
  )
  out_ref = jax.new_ref(jnp.zeros((OUT_D * B,), jnp.float32))
  run_varlen(vidx_ch, dense_flat, vt_pad, out_ref)
  run_sparse(stT, tail_aux, offs.reshape(-1), keys.reshape(-1), out_ref)
  return out_ref[...].reshape(OUT_D, B).T
